# baseline (speedup 1.0000x reference)
import functools

import jax
import jax.numpy as jnp
from jax import lax
from jax.experimental import pallas as pl
from jax.experimental.pallas import tpu as pltpu
from jax.experimental.pallas import tpu_sc as plsc

_B = 16384
_D = 32
_K = 5
_CLIP = 10.0

_NC = 2
_NS = 16
_L = 16
_NW = _NC * _NS
_BW = _B // _NW
_CH = 128


def _sc_scores(center_ids, context_ids, neg_ids_flat, center_emb, context_emb):
  mesh = plsc.VectorSubcoreMesh(core_axis_name="c", subcore_axis_name="s")

  @functools.partial(
      pl.kernel,
      mesh=mesh,
      compiler_params=pltpu.CompilerParams(
          needs_layout_passes=False, use_tc_tiling_on_sc=False),
      out_type=[
          jax.ShapeDtypeStruct((_B,), jnp.float32),
          jax.ShapeDtypeStruct((_B * _K,), jnp.float32),
      ],
      scratch_types=[
          pltpu.VMEM((_BW,), jnp.int32),
          pltpu.VMEM((_BW,), jnp.int32),
          pltpu.VMEM((_BW * _K,), jnp.int32),
          pltpu.VMEM((_BW, _D), jnp.float32),
          pltpu.VMEM((_BW, _D), jnp.float32),
          pltpu.VMEM((_BW * _K, _D), jnp.float32),
          pltpu.VMEM((_BW,), jnp.float32),
          pltpu.VMEM((_BW * _K,), jnp.float32),
          pltpu.SemaphoreType.DMA,
      ],
  )
  def body(cid_hbm, xid_hbm, nid_hbm, cemb_hbm, xemb_hbm,
           pos_hbm, neg_hbm,
           idx_c, idx_x, idx_n, rows_c, rows_x, rows_n, pos_v, neg_v, sem):
    wid = lax.axis_index("s") * _NC + lax.axis_index("c")
    base = wid * _BW
    nbase = wid * (_BW * _K)

    pltpu.sync_copy(cid_hbm.at[pl.ds(base, _BW)], idx_c)
    pltpu.sync_copy(xid_hbm.at[pl.ds(base, _BW)], idx_x)
    pltpu.sync_copy(nid_hbm.at[pl.ds(nbase, _BW * _K)], idx_n)

    copies = []
    for j in range(_BW // _CH):
      sl = pl.ds(j * _CH, _CH)
      copies.append(pltpu.async_copy(cemb_hbm.at[idx_c.at[sl]], rows_c.at[sl], sem))
      copies.append(pltpu.async_copy(xemb_hbm.at[idx_x.at[sl]], rows_x.at[sl], sem))
    for j in range(_BW * _K // _CH):
      sl = pl.ds(j * _CH, _CH)
      copies.append(pltpu.async_copy(xemb_hbm.at[idx_n.at[sl]], rows_n.at[sl], sem))
    for c in copies:
      c.wait()

    iota = lax.iota(jnp.int32, _L)

    def group(g, carry):
      b_vec = g * _L + iota
      b5 = b_vec * _K
      acc_p = jnp.zeros((_L,), jnp.float32)
      accs = [jnp.zeros((_L,), jnp.float32) for _ in range(_K)]
      for d in range(_D):
        dcol = jnp.full((_L,), d, jnp.int32)
        c_col = plsc.load_gather(rows_c, [b_vec, dcol])
        x_col = plsc.load_gather(rows_x, [b_vec, dcol])
        acc_p = acc_p + c_col * x_col
        for k in range(_K):
          n_col = plsc.load_gather(rows_n, [b5 + k, dcol])
          accs[k] = accs[k] + c_col * n_col
      pos_v[pl.ds(g * _L, _L)] = acc_p
      for k in range(_K):
        neg_v[pl.ds(k * _BW + g * _L, _L)] = accs[k]
      return carry

    lax.fori_loop(0, _BW // _L, group, 0)

    pltpu.sync_copy(pos_v, pos_hbm.at[pl.ds(base, _BW)])
    pltpu.sync_copy(neg_v, neg_hbm.at[pl.ds(nbase, _BW * _K)])

  return body(center_ids, context_ids, neg_ids_flat, center_emb, context_emb)


def _finish_body(pos_ref, neg_ref, out_ref):
  p = jnp.clip(pos_ref[...], -_CLIP, _CLIP)
  n = jnp.clip(neg_ref[...], -_CLIP, _CLIP)
  lp = jnp.maximum(-p, 0.0) + jnp.log1p(jnp.exp(-jnp.abs(p)))
  ln = jnp.maximum(n, 0.0) + jnp.log1p(jnp.exp(-jnp.abs(n)))
  total = jnp.sum(lp) + jnp.sum(ln)
  out_ref[...] = jnp.reshape(total * (1.0 / _B), (1, 1))


def kernel(center_ids, context_ids, neg_context_ids, center_emb, context_emb):
  cid = center_ids.astype(jnp.int32)
  xid = context_ids.astype(jnp.int32)
  nid = neg_context_ids.astype(jnp.int32).reshape(-1)
  pos, neg = _sc_scores(cid, xid, nid, center_emb, context_emb)
  out = pl.pallas_call(
      _finish_body,
      out_shape=jax.ShapeDtypeStruct((1, 1), jnp.float32),
  )(pos.reshape(_B // 128, 128), neg.reshape(_B * _K // 128, 128))
  return out[0, 0]
